# baseline (speedup 1.0000x reference)
import functools

import jax
import jax.numpy as jnp
from jax import lax
from jax.experimental import pallas as pl
from jax.experimental.pallas import tpu as pltpu
from jax.experimental.pallas import tpu_sc as plsc

N = 10000
D = 128
HD = D // 2
E = 320000
NC, NS = 2, 16
CH = 128
K = 160
EPT = CH * K
EPAD = NS * EPT
NP = 10240
RPT = NP // NS
HPT = N // NS
G = 2
NG = K // G
NB = 2 * G
DW = 1024
NWD = EPAD // (NC * NS * DW)

_mesh = plsc.VectorSubcoreMesh(
    core_axis_name="c", subcore_axis_name="s", num_cores=NC, num_subcores=NS
)
_sc_params = pltpu.CompilerParams(
    needs_layout_passes=False, use_tc_tiling_on_sc=False
)


@functools.partial(
    pl.kernel,
    out_type=jax.ShapeDtypeStruct((NC, NS, NP), jnp.float32),
    mesh=_mesh,
    scratch_types=[
        pltpu.VMEM((2, DW), jnp.int32),
        pltpu.VMEM((NP,), jnp.float32),
        pltpu.SemaphoreType.DMA,
    ],
    compiler_params=_sc_params,
)
def _sc_deg(dst_hbm, degp_hbm, didx_v, hist_v, sem):
    cid = lax.axis_index("c")
    sid = lax.axis_index("s")

    pltpu.async_copy(dst_hbm.at[cid, sid, 0], didx_v.at[0], sem)
    pltpu.async_copy(dst_hbm.at[cid, sid, 1], didx_v.at[1], sem)

    def hinit(i, carry):
        hist_v[pl.ds(16 * i, 16)] = jnp.zeros((16,), jnp.float32)
        return carry

    lax.fori_loop(0, NP // 16, hinit, 0)

    ones16 = jnp.ones((16,), jnp.float32)

    def wait_w():
        pltpu.make_async_copy(dst_hbm.at[0, 0, 0], didx_v.at[0], sem).wait()

    def body(w, carry):
        wait_w()

        def add16(l, carry2):
            idx = didx_v[w % 2, pl.ds(16 * l, 16)]
            plsc.addupdate_scatter(hist_v, [idx], ones16)
            return carry2

        lax.fori_loop(0, DW // 16, add16, 0)

        @pl.when(w + 2 < NWD)
        def _():
            pltpu.async_copy(dst_hbm.at[cid, sid, w + 2],
                             didx_v.at[(w + 2) % 2], sem)

        return carry

    lax.fori_loop(0, NWD, body, 0)
    pltpu.sync_copy(hist_v, degp_hbm.at[cid, sid])


@functools.partial(
    pl.kernel,
    out_type=jax.ShapeDtypeStruct((NC, NP, HD), jnp.float32),
    mesh=_mesh,
    scratch_types=[
        pltpu.VMEM((3, G * CH), jnp.int32),
        pltpu.VMEM((3, G, CH), jnp.int32),
        pltpu.VMEM((2, CH), jnp.int32),
        pltpu.VMEM((NB, CH, HD), jnp.float32),
        pltpu.VMEM_SHARED((N, HD), jnp.float32),
        pltpu.VMEM_SHARED((NP, HD), jnp.float32),
        pltpu.SemaphoreType.DMA,
        pltpu.SemaphoreType.DMA,
        pltpu.SemaphoreType.DMA,
        pltpu.SemaphoreType.DMA,
    ],
    compiler_params=_sc_params,
)
def _sc_edge(htab_hbm, htabf_hbm, src_hbm, dst_hbm, part_hbm,
             sidx_v, didx_v, hbmidx_v, rows_v, htab_sh, acc_sh,
             sem_i, sem_g, sem_h, sem_s):
    cid = lax.axis_index("c")
    sid = lax.axis_index("s")
    r0 = sid * RPT

    def fire_idx(g):
        p = g % 3
        pltpu.async_copy(src_hbm.at[sid, g], sidx_v.at[p], sem_i)
        pltpu.async_copy(dst_hbm.at[sid, g], didx_v.at[p], sem_i)

    def wait_idx():
        pltpu.make_async_copy(src_hbm.at[0, 0], sidx_v.at[0], sem_i).wait()
        pltpu.make_async_copy(dst_hbm.at[0, 0], didx_v.at[0], sem_i).wait()

    def fire_gathers(g):
        p, base = g % 3, (g % 2) * G
        use_hbm = (g % 4) != 3
        pltpu.async_copy(
            htab_sh.at[sidx_v.at[p, pl.ds(0, CH)]], rows_v.at[base], sem_g
        )

        @pl.when(use_hbm)
        def _():
            hs = g % 2
            off = jnp.full((16,), N, jnp.int32) * cid
            for l in range(CH // 16):
                v = sidx_v[p, pl.ds(CH + 16 * l, 16)]
                hbmidx_v[hs, pl.ds(16 * l, 16)] = v + off
            pltpu.async_copy(
                htabf_hbm.at[hbmidx_v.at[hs]], rows_v.at[base + 1], sem_h
            )

        @pl.when(jnp.logical_not(use_hbm))
        def _():
            pltpu.async_copy(
                htab_sh.at[sidx_v.at[p, pl.ds(CH, CH)]],
                rows_v.at[base + 1], sem_g,
            )

    def wait_gather():
        pltpu.make_async_copy(
            htab_sh.at[sidx_v.at[0, pl.ds(0, CH)]], rows_v.at[0], sem_g
        ).wait()

    def wait_hgather():
        pltpu.make_async_copy(
            htabf_hbm.at[hbmidx_v.at[0]], rows_v.at[0], sem_h
        ).wait()

    def wait_scatter():
        pltpu.make_async_copy(
            rows_v.at[0], acc_sh.at[didx_v.at[0, 0]], sem_s
        ).wait()

    def process(g):
        p, base = g % 3, (g % 2) * G
        use_hbm = (g % 4) != 3
        wait_gather()
        pltpu.async_copy(
            rows_v.at[base], acc_sh.at[didx_v.at[p, 0]], sem_s, add=True
        )

        @pl.when(use_hbm)
        def _():
            wait_hgather()

        @pl.when(jnp.logical_not(use_hbm))
        def _():
            wait_gather()

        pltpu.async_copy(
            rows_v.at[base + 1], acc_sh.at[didx_v.at[p, 1]], sem_s, add=True
        )

    fire_idx(0)
    fire_idx(1)
    fire_idx(2)

    def zinit(i, carry):
        for jj in range(HD // 16):
            rows_v[0, i, pl.ds(16 * jj, 16)] = jnp.zeros((16,), jnp.float32)
        return carry

    lax.fori_loop(0, CH, zinit, 0)

    for k5 in range(RPT // CH):
        pltpu.sync_copy(rows_v.at[0], acc_sh.at[pl.ds(r0 + CH * k5, CH)])
    pltpu.sync_copy(htab_hbm.at[cid, pl.ds(sid * HPT, HPT)],
                    htab_sh.at[pl.ds(sid * HPT, HPT)])
    plsc.subcore_barrier()

    wait_idx()
    fire_gathers(jnp.int32(0))
    wait_idx()
    fire_gathers(jnp.int32(1))
    process(jnp.int32(0))

    def body(g, carry):
        for _ in range(G):
            wait_scatter()
        fire_idx(g + 2)
        wait_idx()
        fire_gathers(g + 1)
        process(g)
        return carry

    lax.fori_loop(1, NG - 2, body, 0)
    for _ in range(G):
        wait_scatter()
    wait_idx()
    fire_gathers(jnp.int32(NG - 1))
    process(jnp.int32(NG - 2))
    for _ in range(G):
        wait_scatter()
    process(jnp.int32(NG - 1))
    for _ in range(G):
        wait_scatter()
    plsc.subcore_barrier()

    pltpu.sync_copy(acc_sh.at[pl.ds(r0, RPT)],
                    part_hbm.at[cid, pl.ds(r0, RPT)])


_BN = 1024


def _tc_body(h_ref, p_ref, d_ref, wt_ref, o_ref):
    neigh = jnp.concatenate([p_ref[0], p_ref[1]], axis=-1)
    deg = jnp.sum(d_ref[...], axis=(0, 1)).reshape(_BN, 1)
    mean = neigh / jnp.maximum(deg, 1.0)
    hb = h_ref[...]
    z = jnp.dot(hb, wt_ref[:D], preferred_element_type=jnp.float32)
    z = z + jnp.dot(mean, wt_ref[D:], preferred_element_type=jnp.float32)
    z = jnp.maximum(z, 0.0)
    o_ref[...] = z / (jnp.sqrt(jnp.sum(z * z, axis=-1, keepdims=True)) + 1e-12)


_tc_layer = pl.pallas_call(
    _tc_body,
    grid=(NP // _BN,),
    in_specs=[
        pl.BlockSpec((_BN, D), lambda i: (i, 0)),
        pl.BlockSpec((NC, _BN, HD), lambda i: (0, i, 0)),
        pl.BlockSpec((NC, NS, _BN), lambda i: (0, 0, i)),
        pl.BlockSpec((2 * D, D), lambda i: (0, 0)),
    ],
    out_specs=pl.BlockSpec((_BN, D), lambda i: (i, 0)),
    out_shape=jax.ShapeDtypeStruct((N, D), jnp.float32),
)


def kernel(x, edge_index, W1, W2):
    src = edge_index[0].astype(jnp.int32)
    dst = edge_index[1].astype(jnp.int32)
    pad = EPAD - E
    src_p = jnp.concatenate([src, jnp.zeros((pad,), jnp.int32)])
    src_w = src_p.reshape(NS, NG, G * CH)
    dst_pad = N + (jnp.arange(pad, dtype=jnp.int32) % (NP - N))
    dst_p = jnp.concatenate([dst, dst_pad])
    dst_w = dst_p.reshape(NS, NG, G, CH)
    dst_d = dst_p.reshape(NC, NS, NWD, DW)

    degp = _sc_deg(dst_d)
    src_w = src_w + (0.0 * degp[0, 0, 0]).astype(jnp.int32)
    h = x
    for W in (W1, W2):
        htab = h.reshape(N, NC, HD).transpose(1, 0, 2)
        part = _sc_edge(htab, htab.reshape(NC * N, HD), src_w, dst_w)
        h = _tc_layer(h, part, degp, W.T)
    return h


# speedup vs baseline: 1.2942x; 1.2942x over previous
import functools

import jax
import jax.numpy as jnp
from jax import lax
from jax.experimental import pallas as pl
from jax.experimental.pallas import tpu as pltpu
from jax.experimental.pallas import tpu_sc as plsc

N = 10000
D = 128
HD = D // 2
E = 320000
NC, NS = 2, 16
CH = 128
K = 160
EPT = CH * K
EPAD = NS * EPT
NP = 10240
RPT = NP // NS
HPT = N // NS
G = 2
NG = K // G
NB = 2 * G
DW = 1024
NWD = EPAD // (NC * NS * DW)

_mesh = plsc.VectorSubcoreMesh(
    core_axis_name="c", subcore_axis_name="s", num_cores=NC, num_subcores=NS
)
_sc_params = pltpu.CompilerParams(
    needs_layout_passes=False, use_tc_tiling_on_sc=False
)


@functools.partial(
    pl.kernel,
    out_type=jax.ShapeDtypeStruct((NC, NS, NP), jnp.float32),
    mesh=_mesh,
    scratch_types=[
        pltpu.VMEM((2, DW), jnp.int32),
        pltpu.VMEM((NP,), jnp.float32),
        pltpu.SemaphoreType.DMA,
    ],
    compiler_params=_sc_params,
)
def _sc_deg(dst_hbm, degp_hbm, didx_v, hist_v, sem):
    cid = lax.axis_index("c")
    sid = lax.axis_index("s")

    pltpu.async_copy(dst_hbm.at[cid, sid, 0], didx_v.at[0], sem)
    pltpu.async_copy(dst_hbm.at[cid, sid, 1], didx_v.at[1], sem)

    def hinit(i, carry):
        hist_v[pl.ds(16 * i, 16)] = jnp.zeros((16,), jnp.float32)
        return carry

    lax.fori_loop(0, NP // 16, hinit, 0)

    ones16 = jnp.ones((16,), jnp.float32)

    def wait_w():
        pltpu.make_async_copy(dst_hbm.at[0, 0, 0], didx_v.at[0], sem).wait()

    def body(w, carry):
        wait_w()

        def add16(l, carry2):
            idx = didx_v[w % 2, pl.ds(16 * l, 16)]
            plsc.addupdate_scatter(hist_v, [idx], ones16)
            return carry2

        lax.fori_loop(0, DW // 16, add16, 0)

        @pl.when(w + 2 < NWD)
        def _():
            pltpu.async_copy(dst_hbm.at[cid, sid, w + 2],
                             didx_v.at[(w + 2) % 2], sem)

        return carry

    lax.fori_loop(0, NWD, body, 0)
    pltpu.sync_copy(hist_v, degp_hbm.at[cid, sid])


@functools.partial(
    pl.kernel,
    out_type=jax.ShapeDtypeStruct((NC, NP, HD), jnp.float32),
    mesh=_mesh,
    scratch_types=[
        pltpu.VMEM((3, G * CH), jnp.int32),
        pltpu.VMEM((3, G, CH), jnp.int32),
        pltpu.VMEM((NB, CH, HD), jnp.float32),
        pltpu.VMEM_SHARED((N, HD), jnp.float32),
        pltpu.VMEM_SHARED((NP, HD), jnp.float32),
        pltpu.SemaphoreType.DMA,
        pltpu.SemaphoreType.DMA,
        pltpu.SemaphoreType.DMA,
    ],
    compiler_params=_sc_params,
)
def _sc_edge(htab_hbm, src_hbm, dst_hbm, part_hbm,
             sidx_v, didx_v, rows_v, htab_sh, acc_sh, sem_i, sem_g, sem_s):
    cid = lax.axis_index("c")
    sid = lax.axis_index("s")
    r0 = sid * RPT

    def fire_idx(g):
        p = g % 3
        pltpu.async_copy(src_hbm.at[sid, g], sidx_v.at[p], sem_i)
        pltpu.async_copy(dst_hbm.at[sid, g], didx_v.at[p], sem_i)

    def wait_idx():
        pltpu.make_async_copy(src_hbm.at[0, 0], sidx_v.at[0], sem_i).wait()
        pltpu.make_async_copy(dst_hbm.at[0, 0], didx_v.at[0], sem_i).wait()

    def fire_gathers(g):
        p, base = g % 3, (g % 2) * G
        for b in range(G):
            pltpu.async_copy(
                htab_sh.at[sidx_v.at[p, pl.ds(b * CH, CH)]],
                rows_v.at[base + b], sem_g,
            )

    def wait_gather():
        pltpu.make_async_copy(
            htab_sh.at[sidx_v.at[0, pl.ds(0, CH)]], rows_v.at[0], sem_g
        ).wait()

    def wait_scatter():
        pltpu.make_async_copy(
            rows_v.at[0], acc_sh.at[didx_v.at[0, 0]], sem_s
        ).wait()

    def process(g):
        p, base = g % 3, (g % 2) * G
        for b in range(G):
            wait_gather()
            pltpu.async_copy(
                rows_v.at[base + b], acc_sh.at[didx_v.at[p, b]],
                sem_s, add=True,
            )

    fire_idx(0)
    fire_idx(1)
    fire_idx(2)

    def zinit(i, carry):
        for jj in range(HD // 16):
            rows_v[0, i, pl.ds(16 * jj, 16)] = jnp.zeros((16,), jnp.float32)
        return carry

    lax.fori_loop(0, CH, zinit, 0)

    for k5 in range(RPT // CH):
        pltpu.sync_copy(rows_v.at[0], acc_sh.at[pl.ds(r0 + CH * k5, CH)])
    pltpu.sync_copy(htab_hbm.at[cid, pl.ds(sid * HPT, HPT)],
                    htab_sh.at[pl.ds(sid * HPT, HPT)])
    plsc.subcore_barrier()

    wait_idx()
    fire_gathers(0)
    wait_idx()
    fire_gathers(1)
    process(0)

    def body(g, carry):
        for _ in range(G):
            wait_scatter()
        fire_idx(g + 2)
        wait_idx()
        fire_gathers(g + 1)
        process(g)
        return carry

    lax.fori_loop(1, NG - 2, body, 0)
    for _ in range(G):
        wait_scatter()
    wait_idx()
    fire_gathers(NG - 1)
    process(NG - 2)
    for _ in range(G):
        wait_scatter()
    process(NG - 1)
    for _ in range(G):
        wait_scatter()
    plsc.subcore_barrier()

    pltpu.sync_copy(acc_sh.at[pl.ds(r0, RPT)],
                    part_hbm.at[cid, pl.ds(r0, RPT)])


_BN = 1024


def _tc_body(h_ref, p_ref, d_ref, wt_ref, o_ref):
    neigh = jnp.concatenate([p_ref[0], p_ref[1]], axis=-1)
    deg = jnp.sum(d_ref[...], axis=(0, 1)).reshape(_BN, 1)
    mean = neigh / jnp.maximum(deg, 1.0)
    hb = h_ref[...]
    z = jnp.dot(hb, wt_ref[:D], preferred_element_type=jnp.float32)
    z = z + jnp.dot(mean, wt_ref[D:], preferred_element_type=jnp.float32)
    z = jnp.maximum(z, 0.0)
    o_ref[...] = z / (jnp.sqrt(jnp.sum(z * z, axis=-1, keepdims=True)) + 1e-12)


_tc_layer = pl.pallas_call(
    _tc_body,
    grid=(NP // _BN,),
    in_specs=[
        pl.BlockSpec((_BN, D), lambda i: (i, 0)),
        pl.BlockSpec((NC, _BN, HD), lambda i: (0, i, 0)),
        pl.BlockSpec((NC, NS, _BN), lambda i: (0, 0, i)),
        pl.BlockSpec((2 * D, D), lambda i: (0, 0)),
    ],
    out_specs=pl.BlockSpec((_BN, D), lambda i: (i, 0)),
    out_shape=jax.ShapeDtypeStruct((N, D), jnp.float32),
)


def kernel(x, edge_index, W1, W2):
    src = edge_index[0].astype(jnp.int32)
    dst = edge_index[1].astype(jnp.int32)
    pad = EPAD - E
    src_p = jnp.concatenate([src, jnp.zeros((pad,), jnp.int32)])
    src_w = src_p.reshape(NS, NG, G * CH)
    dst_pad = N + (jnp.arange(pad, dtype=jnp.int32) % (NP - N))
    dst_p = jnp.concatenate([dst, dst_pad])
    dst_w = dst_p.reshape(NS, NG, G, CH)
    dst_d = dst_p.reshape(NC, NS, NWD, DW)

    degp = _sc_deg(dst_d)
    src_w = src_w + (0.0 * degp[0, 0, 0]).astype(jnp.int32)
    h = x
    for W in (W1, W2):
        htab = h.reshape(N, NC, HD).transpose(1, 0, 2)
        part = _sc_edge(htab, src_w, dst_w)
        h = _tc_layer(h, part, degp, W.T)
    return h


# speedup vs baseline: 1.3064x; 1.0095x over previous
import functools

import jax
import jax.numpy as jnp
from jax import lax
from jax.experimental import pallas as pl
from jax.experimental.pallas import tpu as pltpu
from jax.experimental.pallas import tpu_sc as plsc

N = 10000
D = 128
HD = D // 2
E = 320000
NC, NS = 2, 16
CH = 128
K = 160
EPT = CH * K
EPAD = NS * EPT
NP = 10240
RPT = NP // NS
HPT = N // NS
G = 2
NG = K // G
NB = 2 * G
DW = 1024
NWD = EPAD // (NC * NS * DW)

_mesh = plsc.VectorSubcoreMesh(
    core_axis_name="c", subcore_axis_name="s", num_cores=NC, num_subcores=NS
)
_sc_params = pltpu.CompilerParams(
    needs_layout_passes=False, use_tc_tiling_on_sc=False
)


@functools.partial(
    pl.kernel,
    out_type=jax.ShapeDtypeStruct((NC, NS, NP), jnp.float32),
    mesh=_mesh,
    scratch_types=[
        pltpu.VMEM((2, DW), jnp.int32),
        pltpu.VMEM((NP,), jnp.float32),
        pltpu.SemaphoreType.DMA,
    ],
    compiler_params=_sc_params,
)
def _sc_deg(dst_hbm, degp_hbm, didx_v, hist_v, sem):
    cid = lax.axis_index("c")
    sid = lax.axis_index("s")

    pltpu.async_copy(dst_hbm.at[cid, sid, 0], didx_v.at[0], sem)
    pltpu.async_copy(dst_hbm.at[cid, sid, 1], didx_v.at[1], sem)

    def hinit(i, carry):
        hist_v[pl.ds(16 * i, 16)] = jnp.zeros((16,), jnp.float32)
        return carry

    lax.fori_loop(0, NP // 16, hinit, 0)

    ones16 = jnp.ones((16,), jnp.float32)

    def wait_w():
        pltpu.make_async_copy(dst_hbm.at[0, 0, 0], didx_v.at[0], sem).wait()

    def body(w, carry):
        wait_w()

        def add16(l, carry2):
            idx = didx_v[w % 2, pl.ds(16 * l, 16)]
            plsc.addupdate_scatter(hist_v, [idx], ones16)
            return carry2

        lax.fori_loop(0, DW // 16, add16, 0)

        @pl.when(w + 2 < NWD)
        def _():
            pltpu.async_copy(dst_hbm.at[cid, sid, w + 2],
                             didx_v.at[(w + 2) % 2], sem)

        return carry

    lax.fori_loop(0, NWD, body, 0)
    pltpu.sync_copy(hist_v, degp_hbm.at[cid, sid])


@functools.partial(
    pl.kernel,
    out_type=jax.ShapeDtypeStruct((NC, NP, HD), jnp.float32),
    mesh=_mesh,
    scratch_types=[
        pltpu.VMEM((3, G * CH), jnp.int32),
        pltpu.VMEM((3, G, CH), jnp.int32),
        pltpu.VMEM((NB, CH, HD), jnp.float32),
        pltpu.VMEM_SHARED((N, HD), jnp.float32),
        pltpu.VMEM_SHARED((NP, HD), jnp.float32),
        pltpu.SemaphoreType.DMA,
        pltpu.SemaphoreType.DMA,
        pltpu.SemaphoreType.DMA,
    ],
    compiler_params=_sc_params,
)
def _sc_edge(htab_hbm, src_hbm, dst_hbm, part_hbm,
             sidx_v, didx_v, rows_v, htab_sh, acc_sh, sem_i, sem_g, sem_s):
    cid = lax.axis_index("c")
    sid = lax.axis_index("s")
    r0 = sid * RPT

    def fire_idx(g):
        p = g % 3
        pltpu.async_copy(src_hbm.at[sid, g], sidx_v.at[p], sem_i)
        pltpu.async_copy(dst_hbm.at[sid, g], didx_v.at[p], sem_i)

    def wait_idx():
        pltpu.make_async_copy(src_hbm.at[0, 0], sidx_v.at[0], sem_i).wait()
        pltpu.make_async_copy(dst_hbm.at[0, 0], didx_v.at[0], sem_i).wait()

    def fire_gathers(g):
        p, base = g % 3, (g % 2) * G
        for b in range(G):
            pltpu.async_copy(
                htab_sh.at[sidx_v.at[p, pl.ds(b * CH, CH)]],
                rows_v.at[base + b], sem_g,
            )

    def wait_gather():
        pltpu.make_async_copy(
            htab_sh.at[sidx_v.at[0, pl.ds(0, CH)]], rows_v.at[0], sem_g
        ).wait()

    def wait_scatter():
        pltpu.make_async_copy(
            rows_v.at[0], acc_sh.at[didx_v.at[0, 0]], sem_s
        ).wait()

    def process(g):
        p, base = g % 3, (g % 2) * G
        for b in range(G):
            wait_gather()
            pltpu.async_copy(
                rows_v.at[base + b], acc_sh.at[didx_v.at[p, b]],
                sem_s, add=True,
            )

    fire_idx(0)
    fire_idx(1)
    fire_idx(2)

    def zinit(i, carry):
        for jj in range(HD // 16):
            rows_v[0, i, pl.ds(16 * jj, 16)] = jnp.zeros((16,), jnp.float32)
        return carry

    lax.fori_loop(0, CH, zinit, 0)

    for k5 in range(RPT // CH):
        pltpu.sync_copy(rows_v.at[0], acc_sh.at[pl.ds(r0 + CH * k5, CH)])
    pltpu.sync_copy(htab_hbm.at[cid, pl.ds(sid * HPT, HPT)],
                    htab_sh.at[pl.ds(sid * HPT, HPT)])
    plsc.subcore_barrier()

    wait_idx()
    fire_gathers(0)
    wait_idx()
    fire_gathers(1)
    process(0)

    def body(g, carry):
        for _ in range(G):
            wait_scatter()
        fire_idx(g + 2)
        wait_idx()
        fire_gathers(g + 1)
        process(g)
        return carry

    lax.fori_loop(1, NG - 2, body, 0)
    for _ in range(G):
        wait_scatter()
    wait_idx()
    fire_gathers(NG - 1)
    process(NG - 2)
    for _ in range(G):
        wait_scatter()
    process(NG - 1)
    for _ in range(G):
        wait_scatter()
    plsc.subcore_barrier()

    pltpu.sync_copy(acc_sh.at[pl.ds(r0, RPT)],
                    part_hbm.at[cid, pl.ds(r0, RPT)])


_BN = 2048


def _tc_body(h_ref, p_ref, d_ref, wt_ref, o_ref):
    neigh = jnp.concatenate([p_ref[0], p_ref[1]], axis=-1)
    deg = jnp.sum(d_ref[...], axis=(0, 1)).reshape(_BN, 1)
    mean = neigh / jnp.maximum(deg, 1.0)
    hb = h_ref[...]
    z = jnp.dot(hb, wt_ref[:D], preferred_element_type=jnp.float32)
    z = z + jnp.dot(mean, wt_ref[D:], preferred_element_type=jnp.float32)
    z = jnp.maximum(z, 0.0)
    o_ref[...] = z / (jnp.sqrt(jnp.sum(z * z, axis=-1, keepdims=True)) + 1e-12)


_tc_layer = pl.pallas_call(
    _tc_body,
    grid=(NP // _BN,),
    in_specs=[
        pl.BlockSpec((_BN, D), lambda i: (i, 0)),
        pl.BlockSpec((NC, _BN, HD), lambda i: (0, i, 0)),
        pl.BlockSpec((NC, NS, _BN), lambda i: (0, 0, i)),
        pl.BlockSpec((2 * D, D), lambda i: (0, 0)),
    ],
    out_specs=pl.BlockSpec((_BN, D), lambda i: (i, 0)),
    out_shape=jax.ShapeDtypeStruct((N, D), jnp.float32),
)


def kernel(x, edge_index, W1, W2):
    src = edge_index[0].astype(jnp.int32)
    dst = edge_index[1].astype(jnp.int32)
    pad = EPAD - E
    src_p = jnp.concatenate([src, jnp.zeros((pad,), jnp.int32)])
    src_w = src_p.reshape(NS, NG, G * CH)
    dst_pad = N + (jnp.arange(pad, dtype=jnp.int32) % (NP - N))
    dst_p = jnp.concatenate([dst, dst_pad])
    dst_w = dst_p.reshape(NS, NG, G, CH)
    dst_d = dst_p.reshape(NC, NS, NWD, DW)

    degp = _sc_deg(dst_d)
    src_w = src_w + (0.0 * degp[0, 0, 0]).astype(jnp.int32)
    h = x
    for W in (W1, W2):
        htab = h.reshape(N, NC, HD).transpose(1, 0, 2)
        part = _sc_edge(htab, src_w, dst_w)
        h = _tc_layer(h, part, degp, W.T)
    return h
